# Initial kernel scaffold; baseline (speedup 1.0000x reference)
import jax
import jax.numpy as jnp
from jax.experimental import pallas as pl


def kernel(x, edge_index, W1, b1, W2, b2):
    raise NotImplementedError("write your pallas kernel here")


# speedup vs baseline: 9.7913x; 9.7913x over previous
import functools

import jax
import jax.numpy as jnp
from jax import lax
from jax.experimental import pallas as pl
from jax.experimental.pallas import tpu as pltpu
from jax.experimental.pallas import tpu_sc as plsc

N = 10000
NH = 5120
NPH = 5248
F = 128
E = 320000
NW = 32
EPW = E // NW
CH = 125
NCH = EPW // CH
RPT = NPH // 16
NPD = 10240
DRT = NPD // 16
DRC = 128
DW = 16
BR = 1000


def _mesh():
    return plsc.VectorSubcoreMesh(core_axis_name="c", subcore_axis_name="s")


def _sc_deg(dstm):
    @functools.partial(
        pl.kernel,
        mesh=_mesh(),
        out_type=jax.ShapeDtypeStruct((2, NPD, DW), jnp.float32),
        scratch_types=[
            pltpu.VMEM((NCH, CH), jnp.int32),
            pltpu.VMEM((CH, DW), jnp.float32),
            pltpu.VMEM((DRC, DW), jnp.float32),
            pltpu.VMEM_SHARED((NPD, DW), jnp.float32),
        ],
    )
    def k(dst_hbm, out_hbm, dst_v, ones_v, stage_v, acc_sh):
        cid = lax.axis_index("c")
        sid = lax.axis_index("s")
        wid = sid * 2 + cid

        def fill_ones(i, _):
            ones_v[i, :] = jnp.full((DW,), 1.0, jnp.float32)
            return 0

        lax.fori_loop(0, CH, fill_ones, 0)

        def fill_zero(i, _):
            stage_v[i, :] = jnp.zeros((DW,), jnp.float32)
            return 0

        lax.fori_loop(0, DRC, fill_zero, 0)
        for r in range(DRT // DRC):
            pltpu.sync_copy(stage_v, acc_sh.at[pl.ds(sid * DRT + r * DRC, DRC)])
        plsc.subcore_barrier()

        pltpu.sync_copy(dst_hbm.at[pl.ds(wid * NCH, NCH)], dst_v)

        def body(j, _):
            pltpu.sync_copy(ones_v, acc_sh.at[dst_v.at[j]], add=True)
            return 0

        lax.fori_loop(0, NCH, body, 0)
        plsc.subcore_barrier()

        for r in range(DRT // DRC):
            row0 = sid * DRT + r * DRC
            pltpu.sync_copy(acc_sh.at[pl.ds(row0, DRC)], stage_v)
            pltpu.sync_copy(stage_v, out_hbm.at[cid, pl.ds(row0, DRC)])

    return k(dstm)


def _sc_agg(hs, srcm, dstm):
    @functools.partial(
        pl.kernel,
        mesh=_mesh(),
        out_type=jax.ShapeDtypeStruct((2, NPH, F), jnp.float32),
        scratch_types=[
            pltpu.VMEM((NCH, CH), jnp.int32),
            pltpu.VMEM((NCH, CH), jnp.int32),
            pltpu.VMEM((CH, F), jnp.float32),
            pltpu.VMEM((RPT, F), jnp.float32),
            pltpu.VMEM_SHARED((NPH, F), jnp.float32),
            pltpu.SemaphoreType.DMA,
        ],
    )
    def k(hs_hbm, src_hbm, dst_hbm, out_hbm, src_v, dst_v, rows_v, stage_v,
          acc_sh, sem):
        cid = lax.axis_index("c")
        sid = lax.axis_index("s")
        wid = sid * 2 + cid

        def fill_zero(t, _):
            i = t // (F // 16)
            kk = t % (F // 16)
            stage_v[i, pl.ds(kk * 16, 16)] = jnp.zeros((16,), jnp.float32)
            return 0

        lax.fori_loop(0, RPT * (F // 16), fill_zero, 0)
        pltpu.sync_copy(stage_v, acc_sh.at[pl.ds(sid * RPT, RPT)])
        plsc.subcore_barrier()

        pltpu.sync_copy(src_hbm.at[pl.ds(wid * NCH, NCH)], src_v)
        pltpu.sync_copy(dst_hbm.at[pl.ds(wid * NCH, NCH)], dst_v)

        def body(j, _):
            pltpu.async_copy(hs_hbm.at[src_v.at[j]], rows_v, sem).wait()
            pltpu.sync_copy(rows_v, acc_sh.at[dst_v.at[j]], add=True)
            return 0

        lax.fori_loop(0, NCH, body, 0)
        plsc.subcore_barrier()

        pltpu.sync_copy(acc_sh.at[pl.ds(sid * RPT, RPT)], stage_v)
        pltpu.sync_copy(stage_v, out_hbm.at[cid, pl.ds(sid * RPT, RPT)])

    return k(hs, srcm, dstm)


def _tca_body(x_ref, w_ref, d0_ref, d1_ref, hs_ref, dinv_ref):
    deg = d0_ref[:, 0:1] + d1_ref[:, 0:1] + 1.0
    dinv = lax.rsqrt(deg)
    h = jnp.dot(x_ref[...], w_ref[...], preferred_element_type=jnp.float32)
    hs_ref[...] = h * dinv
    dinv_ref[...] = jnp.broadcast_to(dinv, (BR, F))


def _tc_a(x, W1, d0, d1):
    return pl.pallas_call(
        _tca_body,
        grid=(N // BR,),
        in_specs=[
            pl.BlockSpec((BR, F), lambda i: (i, 0)),
            pl.BlockSpec((F, F), lambda i: (0, 0)),
            pl.BlockSpec((BR, DW), lambda i: (i, 0)),
            pl.BlockSpec((BR, DW), lambda i: (i, 0)),
        ],
        out_specs=[
            pl.BlockSpec((BR, F), lambda i: (i, 0)),
            pl.BlockSpec((BR, F), lambda i: (i, 0)),
        ],
        out_shape=[
            jax.ShapeDtypeStruct((N, F), jnp.float32),
            jax.ShapeDtypeStruct((N, F), jnp.float32),
        ],
    )(x, W1, d0, d1)


def _tcb_body(a0_ref, a1_ref, hs1_ref, dinv_ref, b_ref, w_ref, hs2_ref):
    z = dinv_ref[...] * (a0_ref[...] + a1_ref[...] + hs1_ref[...]) + b_ref[...]
    act = jnp.where(z >= 0, z, 0.01 * z)
    h2 = jnp.dot(act, w_ref[...], preferred_element_type=jnp.float32)
    hs2_ref[...] = h2 * dinv_ref[...]


def _tc_b(a0, a1, hs1, dinv, b1, W2):
    full = pl.BlockSpec((BR, F), lambda i: (i, 0))
    return pl.pallas_call(
        _tcb_body,
        grid=(N // BR,),
        in_specs=[
            full, full, full, full,
            pl.BlockSpec((1, F), lambda i: (0, 0)),
            pl.BlockSpec((F, F), lambda i: (0, 0)),
        ],
        out_specs=full,
        out_shape=jax.ShapeDtypeStruct((N, F), jnp.float32),
    )(a0, a1, hs1, dinv, b1, W2)


def _tcc_body(a0_ref, a1_ref, hs2_ref, dinv_ref, b_ref, out_ref):
    out_ref[...] = (
        dinv_ref[...] * (a0_ref[...] + a1_ref[...] + hs2_ref[...]) + b_ref[...]
    )


def _tc_c(a0, a1, hs2, dinv, b2):
    full = pl.BlockSpec((BR, F), lambda i: (i, 0))
    return pl.pallas_call(
        _tcc_body,
        grid=(N // BR,),
        in_specs=[
            full, full, full, full,
            pl.BlockSpec((1, F), lambda i: (0, 0)),
        ],
        out_specs=full,
        out_shape=jax.ShapeDtypeStruct((N, F), jnp.float32),
    )(a0, a1, hs2, dinv, b2)


def _agg_layer(hs, src, dst0, dst1):
    g0 = _sc_agg(hs, src, dst0)
    g1 = _sc_agg(hs, src, dst1)
    p0 = jnp.concatenate([g0[0, :NH], g1[0, : N - NH]], axis=0)
    p1 = jnp.concatenate([g0[1, :NH], g1[1, : N - NH]], axis=0)
    return p0, p1


def kernel(x, edge_index, W1, b1, W2, b2):
    src = edge_index[0].astype(jnp.int32)
    dst = edge_index[1].astype(jnp.int32)
    srcm = src.reshape(E // CH, CH)
    dstm = dst.reshape(E // CH, CH)
    dst0 = jnp.where(dst < NH, dst, NPH - 1).reshape(E // CH, CH)
    dst1 = jnp.where(dst >= NH, dst - NH, NPH - 1).reshape(E // CH, CH)
    b1r = b1.reshape(1, F)
    b2r = b2.reshape(1, F)

    degp = _sc_deg(dstm)
    hs1, dinv = _tc_a(x, W1, degp[0, :N], degp[1, :N])
    p0, p1 = _agg_layer(hs1, srcm, dst0, dst1)
    hs2 = _tc_b(p0, p1, hs1, dinv, b1r, W2)
    q0, q1 = _agg_layer(hs2, srcm, dst0, dst1)
    return _tc_c(q0, q1, hs2, dinv, b2r)
